# baseline (speedup 1.0000x reference)
import functools

import jax
import jax.numpy as jnp
from jax import lax
from jax.experimental import pallas as pl
from jax.experimental.pallas import tpu as pltpu

_F32 = jnp.float32
_BF16 = jnp.bfloat16


def _gram_kernel(w_ref, b_ref, m_ref, aux_ref):
    w = w_ref[...]
    b = b_ref[...]
    m_ref[...] = lax.dot_general(w, w, (((1,), (1,)), ((), ())),
                                 preferred_element_type=_F32)
    bw = lax.dot_general(b, w, (((1,), (1,)), ((), ())),
                         preferred_element_type=_F32)
    bb = jnp.sum(b * b)
    r = lax.broadcasted_iota(jnp.int32, bw.shape, 0)
    c = lax.broadcasted_iota(jnp.int32, bw.shape, 1)
    aux_ref[...] = bw + jnp.where((r == 1) & (c == 0), bb, 0.0)


def _l2norm(z):
    ssq = jnp.sum(z * z, axis=-1, keepdims=True)
    return z * lax.rsqrt(jnp.maximum(ssq, 1e-24))


def _fused_kernel(adj_ref, feat_ref, wt_ref, wb_ref, w128_ref, b_ref, m_ref,
                  aux_ref, wsel_ref, bsel_ref, ypred_ref, readout_ref,
                  *, N, hidden, emb, K, L, G):
    GN = G * N
    GK = G * K
    E = emb
    adj = adj_ref[0]
    featb = feat_ref[0]

    deg = jnp.sum(adj.astype(_F32), axis=2, keepdims=True)
    recip = (1.0 / jnp.maximum(deg, 1.0)).reshape(GN, 1)

    def agg_b(hb):
        parts = [jnp.dot(adj[g], hb[g * N:(g + 1) * N],
                         preferred_element_type=_F32) for g in range(G)]
        return jnp.concatenate(parts, axis=0) * recip

    def sage(hb, i, dout, relu):
        nb = agg_b(hb).astype(_BF16)
        z = (jnp.dot(hb, wt_ref[:, i * hidden:i * hidden + dout],
                     preferred_element_type=_F32)
             + jnp.dot(nb, wb_ref[:, i * hidden:i * hidden + dout],
                       preferred_element_type=_F32)
             + b_ref[i:i + 1, 0:dout])
        z = _l2norm(z)
        if relu:
            z = jnp.maximum(z, 0.0)
        return z, nb

    h, _ = sage(featb, 0, hidden, True)
    h, _ = sage(h.astype(_BF16), 1, hidden, True)
    hb = h.astype(_BF16)
    h, _ = sage(hb, 2, emb, False)
    hb = h.astype(_BF16)

    n = agg_b(hb)
    nb = n.astype(_BF16)

    zf = (jnp.dot(hb, wt_ref[:, 3 * hidden:4 * hidden], preferred_element_type=_F32)
          + jnp.dot(nb, wb_ref[:, 3 * hidden:4 * hidden], preferred_element_type=_F32)
          + b_ref[3:4, 0:hidden])
    zf = jnp.maximum(_l2norm(zf), 0.0)

    yh = (jnp.dot(hb, m_ref[0:E, 0:E], preferred_element_type=_F32)
          + jnp.dot(nb, m_ref[E:2 * E, 0:E], preferred_element_type=_F32)
          + 2.0 * aux_ref[0:1, 0:E])
    yn = (jnp.dot(hb, m_ref[0:E, E:2 * E], preferred_element_type=_F32)
          + jnp.dot(nb, m_ref[E:2 * E, E:2 * E], preferred_element_type=_F32)
          + 2.0 * aux_ref[0:1, E:2 * E])
    ssq = (jnp.sum(yh * h, axis=-1, keepdims=True)
           + jnp.sum(yn * n, axis=-1, keepdims=True) + aux_ref[1:2, 0:1])
    scale = lax.rsqrt(jnp.maximum(ssq, 1e-24))

    logits = (jnp.dot(hb, wsel_ref[0, 0:E, :], preferred_element_type=_F32)
              + jnp.dot(nb, wsel_ref[0, E:2 * E, :], preferred_element_type=_F32)
              + bsel_ref[0])
    zp = jnp.maximum(logits * scale, 0.0)
    rg = lax.broadcasted_iota(jnp.int32, (GN, GK), 0) // N
    cg = lax.broadcasted_iota(jnp.int32, (GN, GK), 1) // K
    e = jnp.exp(zp) * (rg == cg).astype(_F32)
    s = e / jnp.sum(e, axis=-1, keepdims=True)
    sb = s.astype(_BF16)

    hpool = lax.dot_general(sb, zf.astype(_BF16), (((0,), (0,)), ((), ())),
                            preferred_element_type=_F32)
    tmp = jnp.concatenate(
        [jnp.dot(adj[g], sb[g * N:(g + 1) * N], preferred_element_type=_F32)
         for g in range(G)], axis=0)
    adjp = lax.dot_general(sb, tmp.astype(_BF16), (((0,), (0,)), ((), ())),
                           preferred_element_type=_F32)
    adjpb = adjp.astype(_BF16)

    def bsage(xb, i, dout):
        hn = jnp.dot(adjpb, xb, preferred_element_type=_F32)
        z = (jnp.dot(hn.astype(_BF16),
                     w128_ref[:, (i - 4) * hidden:(i - 4) * hidden + dout],
                     preferred_element_type=_F32) + b_ref[i:i + 1, 0:dout])
        return jnp.maximum(_l2norm(z), 0.0)

    x = bsage(hpool.astype(_BF16), 4, hidden)
    x = bsage(x.astype(_BF16), 5, hidden)
    xb = x.astype(_BF16)
    x = bsage(xb, 6, emb)

    readout_ref[0] = x

    KL = K * L
    z = jnp.dot(x.astype(_BF16), w128_ref[:, 3 * hidden:3 * hidden + KL],
                preferred_element_type=_F32)
    zr = lax.broadcasted_iota(jnp.int32, (GK, KL), 0) % K
    zc = lax.broadcasted_iota(jnp.int32, (GK, KL), 1) // L
    z = jnp.where(zr == zc, z, 0.0)
    rr = lax.broadcasted_iota(jnp.int32, (KL, L), 0) % L
    rc = lax.broadcasted_iota(jnp.int32, (KL, L), 1)
    rsel = (rr == rc).astype(_F32)
    d = jnp.dot(z, rsel, preferred_element_type=_F32)
    gr = lax.broadcasted_iota(jnp.int32, (GK, G), 0) // K
    gc = lax.broadcasted_iota(jnp.int32, (GK, G), 1)
    gsel = (gr == gc).astype(_F32)
    ypred_ref[0] = (lax.dot_general(gsel, d, (((0,), (0,)), ((), ())),
                                    preferred_element_type=_F32)
                    + b_ref[7:8, 0:L])


def kernel(adj, feat, gcb0_w, gcb0_b, gcb1_w, gcb1_b, gcb2_w, gcb2_b,
           featgc_w, featgc_b, poolgc_w, poolgc_b,
           gca0_w, gca0_b, gca1_w, gca1_b, gca2_w, gca2_b, pred_w, pred_b):
    B, N, Din = feat.shape
    hidden = gcb0_w.shape[1]
    emb = gcb2_w.shape[1]
    BK = poolgc_w.shape[1]
    K = BK // B
    L = pred_w.shape[1]
    E2 = poolgc_w.shape[0]

    G = 1
    for d in (8, 4, 2):
        if B % d == 0:
            G = d
            break
    NG = B // G
    GK = G * K

    bp = jnp.zeros((8, BK), _F32).at[0].set(poolgc_b[0])
    m_mat, aux = pl.pallas_call(
        _gram_kernel,
        out_shape=(jax.ShapeDtypeStruct((E2, E2), _F32),
                   jax.ShapeDtypeStruct((8, E2), _F32)),
    )(poolgc_w, bp)

    w256 = jnp.concatenate([gcb0_w, gcb1_w, gcb2_w, featgc_w], axis=1)
    wt = w256[:Din, :].astype(_BF16)
    wb = w256[Din:, :].astype(_BF16)
    w_cat = pred_w.reshape(K, emb, L).transpose(1, 0, 2).reshape(emb, K * L)
    w_cat_p = jnp.zeros((emb, hidden), _F32).at[:, :K * L].set(w_cat)
    w128 = jnp.concatenate([gca0_w, gca1_w, gca2_w, w_cat_p],
                           axis=1).astype(_BF16)

    b_all = jnp.zeros((8, max(hidden, 128)), _F32)
    for i, b in enumerate([gcb0_b, gcb1_b, gcb2_b, featgc_b,
                           gca0_b, gca1_b, gca2_b, pred_b]):
        b_all = b_all.at[i, :b.shape[1]].set(b[0])

    m_bf = m_mat.astype(_BF16)
    wsel = poolgc_w.reshape(E2, NG, GK).transpose(1, 0, 2).astype(_BF16)
    bsel = poolgc_b.reshape(NG, 1, GK)

    adj4 = adj.reshape(NG, G, N, N).astype(_BF16)
    feat4 = feat.reshape(NG, G * N, Din).astype(_BF16)

    kern = functools.partial(_fused_kernel, N=N, hidden=hidden, emb=emb,
                             K=K, L=L, G=G)

    in_specs = [
        pl.BlockSpec((1, G, N, N), lambda i: (i, 0, 0, 0)),
        pl.BlockSpec((1, G * N, Din), lambda i: (i, 0, 0)),
        pl.BlockSpec(wt.shape, lambda i: (0, 0)),
        pl.BlockSpec(wb.shape, lambda i: (0, 0)),
        pl.BlockSpec(w128.shape, lambda i: (0, 0)),
        pl.BlockSpec(b_all.shape, lambda i: (0, 0)),
        pl.BlockSpec((E2, E2), lambda i: (0, 0)),
        pl.BlockSpec((8, E2), lambda i: (0, 0)),
        pl.BlockSpec((1, E2, GK), lambda i: (i, 0, 0)),
        pl.BlockSpec((1, 1, GK), lambda i: (i, 0, 0)),
    ]
    out_specs = (
        pl.BlockSpec((1, G, L), lambda i: (i, 0, 0)),
        pl.BlockSpec((1, GK, emb), lambda i: (i, 0, 0)),
    )

    ypred, readout = pl.pallas_call(
        kern,
        out_shape=(jax.ShapeDtypeStruct((NG, G, L), _F32),
                   jax.ShapeDtypeStruct((NG, GK, emb), _F32)),
        grid=(NG,),
        in_specs=in_specs,
        out_specs=out_specs,
        compiler_params=pltpu.CompilerParams(dimension_semantics=("parallel",)),
    )(adj4, feat4, wt, wb, w128, b_all, m_bf, aux, wsel, bsel)

    return ypred.reshape(B, L), readout.reshape(B, K * emb)


# speedup vs baseline: 1.0731x; 1.0731x over previous
import functools

import jax
import jax.numpy as jnp
from jax import lax
from jax.experimental import pallas as pl
from jax.experimental.pallas import tpu as pltpu

_F32 = jnp.float32
_BF16 = jnp.bfloat16


def _gram_kernel(w_ref, b_ref, m_ref, aux_ref):
    w = w_ref[...]
    b = b_ref[...]
    m_ref[...] = lax.dot_general(w, w, (((1,), (1,)), ((), ())),
                                 preferred_element_type=_F32)
    bw = lax.dot_general(b, w, (((1,), (1,)), ((), ())),
                         preferred_element_type=_F32)
    bb = jnp.sum(b * b)
    r = lax.broadcasted_iota(jnp.int32, bw.shape, 0)
    c = lax.broadcasted_iota(jnp.int32, bw.shape, 1)
    aux_ref[...] = bw + jnp.where((r == 1) & (c == 0), bb, 0.0)


def _l2norm(z):
    ssq = jnp.sum(z * z, axis=-1, keepdims=True)
    return z * lax.rsqrt(jnp.maximum(ssq, 1e-24))


def _fused_kernel(adj_ref, feat_ref, wt_ref, wb_ref, w128_ref, b_ref, m_ref,
                  aux_ref, wsel_ref, bsel_ref, ypred_ref, readout_ref,
                  *, N, hidden, emb, K, L, G):
    GN = G * N
    GK = G * K
    E = emb
    adjf = adj_ref[0]
    adj = adjf.astype(_BF16)
    featb = feat_ref[0].astype(_BF16)

    deg = jnp.sum(adjf, axis=2, keepdims=True)
    recip = (1.0 / jnp.maximum(deg, 1.0)).reshape(GN, 1)

    def agg_b(hb):
        parts = [jnp.dot(adj[g], hb[g * N:(g + 1) * N],
                         preferred_element_type=_F32) for g in range(G)]
        return jnp.concatenate(parts, axis=0) * recip

    def sage(hb, i, dout, relu):
        nb = agg_b(hb).astype(_BF16)
        z = (jnp.dot(hb, wt_ref[:, i * hidden:i * hidden + dout],
                     preferred_element_type=_F32)
             + jnp.dot(nb, wb_ref[:, i * hidden:i * hidden + dout],
                       preferred_element_type=_F32)
             + b_ref[i:i + 1, 0:dout])
        z = _l2norm(z)
        if relu:
            z = jnp.maximum(z, 0.0)
        return z, nb

    h, _ = sage(featb, 0, hidden, True)
    h, _ = sage(h.astype(_BF16), 1, hidden, True)
    hb = h.astype(_BF16)
    h, _ = sage(hb, 2, emb, False)
    hb = h.astype(_BF16)

    n = agg_b(hb)
    nb = n.astype(_BF16)

    zf = (jnp.dot(hb, wt_ref[:, 3 * hidden:4 * hidden], preferred_element_type=_F32)
          + jnp.dot(nb, wb_ref[:, 3 * hidden:4 * hidden], preferred_element_type=_F32)
          + b_ref[3:4, 0:hidden])
    zf = jnp.maximum(_l2norm(zf), 0.0)

    yh = (jnp.dot(hb, m_ref[0:E, 0:E], preferred_element_type=_F32)
          + jnp.dot(nb, m_ref[E:2 * E, 0:E], preferred_element_type=_F32)
          + 2.0 * aux_ref[0:1, 0:E])
    yn = (jnp.dot(hb, m_ref[0:E, E:2 * E], preferred_element_type=_F32)
          + jnp.dot(nb, m_ref[E:2 * E, E:2 * E], preferred_element_type=_F32)
          + 2.0 * aux_ref[0:1, E:2 * E])
    ssq = (jnp.sum(yh * h, axis=-1, keepdims=True)
           + jnp.sum(yn * n, axis=-1, keepdims=True) + aux_ref[1:2, 0:1])
    scale = lax.rsqrt(jnp.maximum(ssq, 1e-24))

    logits = (jnp.dot(hb, wsel_ref[0, 0:E, :], preferred_element_type=_F32)
              + jnp.dot(nb, wsel_ref[0, E:2 * E, :], preferred_element_type=_F32)
              + bsel_ref[0])
    zp = jnp.maximum(logits * scale, 0.0)
    rg = lax.broadcasted_iota(jnp.int32, (GN, GK), 0) // N
    cg = lax.broadcasted_iota(jnp.int32, (GN, GK), 1) // K
    e = jnp.exp(zp) * (rg == cg).astype(_F32)
    s = e / jnp.sum(e, axis=-1, keepdims=True)
    sb = s.astype(_BF16)

    hpool = lax.dot_general(sb, zf.astype(_BF16), (((0,), (0,)), ((), ())),
                            preferred_element_type=_F32)
    tmp = jnp.concatenate(
        [jnp.dot(adj[g], sb[g * N:(g + 1) * N], preferred_element_type=_F32)
         for g in range(G)], axis=0)
    adjp = lax.dot_general(sb, tmp.astype(_BF16), (((0,), (0,)), ((), ())),
                           preferred_element_type=_F32)
    adjpb = adjp.astype(_BF16)

    def bsage(xb, i, dout):
        hn = jnp.dot(adjpb, xb, preferred_element_type=_F32)
        z = (jnp.dot(hn.astype(_BF16),
                     w128_ref[:, (i - 4) * hidden:(i - 4) * hidden + dout],
                     preferred_element_type=_F32) + b_ref[i:i + 1, 0:dout])
        return jnp.maximum(_l2norm(z), 0.0)

    x = bsage(hpool.astype(_BF16), 4, hidden)
    x = bsage(x.astype(_BF16), 5, hidden)
    xb = x.astype(_BF16)
    x = bsage(xb, 6, emb)

    readout_ref[0] = x

    KL = K * L
    z = jnp.dot(x.astype(_BF16), w128_ref[:, 3 * hidden:3 * hidden + KL],
                preferred_element_type=_F32)
    zr = lax.broadcasted_iota(jnp.int32, (GK, KL), 0) % K
    zc = lax.broadcasted_iota(jnp.int32, (GK, KL), 1) // L
    z = jnp.where(zr == zc, z, 0.0)
    rr = lax.broadcasted_iota(jnp.int32, (KL, L), 0) % L
    rc = lax.broadcasted_iota(jnp.int32, (KL, L), 1)
    rsel = (rr == rc).astype(_F32)
    d = jnp.dot(z, rsel, preferred_element_type=_F32)
    gr = lax.broadcasted_iota(jnp.int32, (GK, G), 0) // K
    gc = lax.broadcasted_iota(jnp.int32, (GK, G), 1)
    gsel = (gr == gc).astype(_F32)
    ypred_ref[0] = (lax.dot_general(gsel, d, (((0,), (0,)), ((), ())),
                                    preferred_element_type=_F32)
                    + b_ref[7:8, 0:L])


def kernel(adj, feat, gcb0_w, gcb0_b, gcb1_w, gcb1_b, gcb2_w, gcb2_b,
           featgc_w, featgc_b, poolgc_w, poolgc_b,
           gca0_w, gca0_b, gca1_w, gca1_b, gca2_w, gca2_b, pred_w, pred_b):
    B, N, Din = feat.shape
    hidden = gcb0_w.shape[1]
    emb = gcb2_w.shape[1]
    BK = poolgc_w.shape[1]
    K = BK // B
    L = pred_w.shape[1]
    E2 = poolgc_w.shape[0]

    G = 1
    for d in (8, 4, 2):
        if B % d == 0:
            G = d
            break
    NG = B // G
    GK = G * K

    bp = jnp.zeros((8, BK), _F32).at[0].set(poolgc_b[0])
    m_mat, aux = pl.pallas_call(
        _gram_kernel,
        out_shape=(jax.ShapeDtypeStruct((E2, E2), _F32),
                   jax.ShapeDtypeStruct((8, E2), _F32)),
    )(poolgc_w, bp)

    w256 = jnp.concatenate([gcb0_w, gcb1_w, gcb2_w, featgc_w], axis=1)
    wt = w256[:Din, :].astype(_BF16)
    wb = w256[Din:, :].astype(_BF16)
    w_cat = pred_w.reshape(K, emb, L).transpose(1, 0, 2).reshape(emb, K * L)
    w_cat_p = jnp.zeros((emb, hidden), _F32).at[:, :K * L].set(w_cat)
    w128 = jnp.concatenate([gca0_w, gca1_w, gca2_w, w_cat_p],
                           axis=1).astype(_BF16)

    b_all = jnp.zeros((8, max(hidden, 128)), _F32)
    for i, b in enumerate([gcb0_b, gcb1_b, gcb2_b, featgc_b,
                           gca0_b, gca1_b, gca2_b, pred_b]):
        b_all = b_all.at[i, :b.shape[1]].set(b[0])

    m_bf = m_mat.astype(_BF16)
    wsel = poolgc_w.reshape(E2, NG, GK).transpose(1, 0, 2).astype(_BF16)
    bsel = poolgc_b.reshape(NG, 1, GK)

    adj4 = adj.reshape(NG, G, N, N)
    feat4 = feat.reshape(NG, G * N, Din)

    kern = functools.partial(_fused_kernel, N=N, hidden=hidden, emb=emb,
                             K=K, L=L, G=G)

    in_specs = [
        pl.BlockSpec((1, G, N, N), lambda i: (i, 0, 0, 0)),
        pl.BlockSpec((1, G * N, Din), lambda i: (i, 0, 0)),
        pl.BlockSpec(wt.shape, lambda i: (0, 0)),
        pl.BlockSpec(wb.shape, lambda i: (0, 0)),
        pl.BlockSpec(w128.shape, lambda i: (0, 0)),
        pl.BlockSpec(b_all.shape, lambda i: (0, 0)),
        pl.BlockSpec((E2, E2), lambda i: (0, 0)),
        pl.BlockSpec((8, E2), lambda i: (0, 0)),
        pl.BlockSpec((1, E2, GK), lambda i: (i, 0, 0)),
        pl.BlockSpec((1, 1, GK), lambda i: (i, 0, 0)),
    ]
    out_specs = (
        pl.BlockSpec((1, G, L), lambda i: (i, 0, 0)),
        pl.BlockSpec((1, GK, emb), lambda i: (i, 0, 0)),
    )

    ypred, readout = pl.pallas_call(
        kern,
        out_shape=(jax.ShapeDtypeStruct((NG, G, L), _F32),
                   jax.ShapeDtypeStruct((NG, GK, emb), _F32)),
        grid=(NG,),
        in_specs=in_specs,
        out_specs=out_specs,
        compiler_params=pltpu.CompilerParams(dimension_semantics=("parallel",)),
    )(adj4, feat4, wt, wb, w128, b_all, m_bf, aux, wsel, bsel)

    return ypred.reshape(B, L), readout.reshape(B, K * emb)
